# baseline (speedup 1.0000x reference)
import functools

import jax
import jax.numpy as jnp
from jax import lax
from jax.experimental import pallas as pl
from jax.experimental.pallas import tpu as pltpu
from jax.experimental.pallas import tpu_sc as plsc

EMBED = 64
BATCH = 16384
VOCAB = 100001
BMV = 4096

NC = 2
NS = 16
NW = NC * NS
PER_W = BATCH // NW
CHUNK = 128
NCHUNK = PER_W // CHUNK


def _pmat_body(tt_ref, wh_ref, p_ref):
    p_ref[...] = lax.dot_general(
        tt_ref[...], wh_ref[...], (((0,), (0,)), ((), ())),
        preferred_element_type=jnp.float32)


def _pmat(tableT, WH2):
    return pl.pallas_call(
        _pmat_body,
        grid=(pl.cdiv(VOCAB, BMV),),
        in_specs=[
            pl.BlockSpec((EMBED, BMV), lambda i: (0, i)),
            pl.BlockSpec((EMBED, 2 * EMBED), lambda i: (0, 0)),
        ],
        out_specs=pl.BlockSpec((BMV, 2 * EMBED), lambda i: (i, 0)),
        out_shape=jax.ShapeDtypeStruct((VOCAB, 2 * EMBED), jnp.float32),
    )(tableT, WH2)


def _combine_chunk(gx_v, gy_v, buf, o_v, bvals, eiotas, bufsplat):
    def body(r, _):
        rsplat = jnp.full((16,), r, dtype=jnp.int32)
        for q in range(EMBED // 16):
            s = q * 16
            val = (gx_v[buf, r, pl.ds(s, 16)]
                   + gy_v[buf, r, pl.ds(EMBED + s, 16)]
                   + bvals[q])
            plsc.store_scatter(o_v, [bufsplat, eiotas[q], rsplat], val)
        return 0

    lax.fori_loop(0, CHUNK, body, 0, unroll=4)


@functools.partial(
    pl.kernel,
    mesh=plsc.VectorSubcoreMesh(core_axis_name="c", subcore_axis_name="s"),
    out_type=jax.ShapeDtypeStruct((EMBED, BATCH), jnp.float32),
    scratch_types=[
        pltpu.VMEM((NCHUNK, CHUNK), jnp.int32),
        pltpu.VMEM((NCHUNK, CHUNK), jnp.int32),
        pltpu.VMEM((2, CHUNK, 2 * EMBED), jnp.float32),
        pltpu.VMEM((2, CHUNK, 2 * EMBED), jnp.float32),
        pltpu.VMEM((2, EMBED, CHUNK), jnp.float32),
        pltpu.VMEM((EMBED,), jnp.float32),
        pltpu.SemaphoreType.DMA,
        pltpu.SemaphoreType.DMA,
        pltpu.SemaphoreType.DMA,
    ],
    compiler_params=pltpu.CompilerParams(use_tc_tiling_on_sc=True,
                                         needs_layout_passes=False),
)
def _gather_combine(idx_hbm, p_hbm, b_hbm, out_hbm, ix_v, iy_v, gx_v, gy_v,
                    o_v, b_v, isem, gsem, osem):
    wid = lax.axis_index("s") * NC + lax.axis_index("c")
    base = wid * PER_W
    pltpu.sync_copy(b_hbm, b_v)
    bvals = tuple(b_v[pl.ds(q * 16, 16)] for q in range(EMBED // 16))
    eiotas = tuple(q * 16 + lax.iota(jnp.int32, 16)
                   for q in range(EMBED // 16))
    icp = [
        pltpu.async_copy(idx_hbm.at[pl.ds(base + j * CHUNK, CHUNK)],
                         ix_v.at[j], isem)
        for j in range(NCHUNK)
    ] + [
        pltpu.async_copy(idx_hbm.at[pl.ds(BATCH + base + j * CHUNK, CHUNK)],
                         iy_v.at[j], isem)
        for j in range(NCHUNK)
    ]
    for c in icp:
        c.wait()

    gets = [(pltpu.async_copy(p_hbm.at[ix_v.at[0]], gx_v.at[0], gsem),
             pltpu.async_copy(p_hbm.at[iy_v.at[0]], gy_v.at[0], gsem))]
    puts = []
    for j in range(NCHUNK):
        buf = j % 2
        if j + 1 < NCHUNK:
            nb = (j + 1) % 2
            gets.append(
                (pltpu.async_copy(p_hbm.at[ix_v.at[j + 1]], gx_v.at[nb], gsem),
                 pltpu.async_copy(p_hbm.at[iy_v.at[j + 1]], gy_v.at[nb], gsem)))
        gets[j][0].wait()
        gets[j][1].wait()
        if j >= 2:
            puts[j - 2].wait()
        bufsplat = jnp.full((16,), buf, dtype=jnp.int32)
        _combine_chunk(gx_v, gy_v, buf, o_v, bvals, eiotas, bufsplat)
        puts.append(pltpu.async_copy(
            o_v.at[buf],
            out_hbm.at[:, pl.ds(base + j * CHUNK, CHUNK)], osem))
    puts[NCHUNK - 2].wait()
    puts[NCHUNK - 1].wait()


def kernel(actions, table, W, b):
    idx = actions.astype(jnp.int32).T.reshape(2 * BATCH)
    WH2 = jnp.concatenate([W[:EMBED], W[EMBED:]], axis=1)
    P = _pmat(table.T, WH2)
    return _gather_combine(idx, P, b).T


# speedup vs baseline: 1.2057x; 1.2057x over previous
import functools

import jax
import jax.numpy as jnp
from jax import lax
from jax.experimental import pallas as pl
from jax.experimental.pallas import tpu as pltpu
from jax.experimental.pallas import tpu_sc as plsc

EMBED = 64
BATCH = 16384
VOCAB = 100001
BMV = 8192

NC = 2
NS = 16
NW = NC * NS
PER_W = BATCH // NW
CHUNK = 128
NCHUNK = PER_W // CHUNK


def _pmat_body(tt_ref, wh_ref, p_ref):
    p_ref[...] = lax.dot_general(
        tt_ref[...], wh_ref[...], (((0,), (0,)), ((), ())),
        preferred_element_type=jnp.float32)


def _pmat(tableT, WH2):
    return pl.pallas_call(
        _pmat_body,
        grid=(pl.cdiv(VOCAB, BMV),),
        in_specs=[
            pl.BlockSpec((EMBED, BMV), lambda i: (0, i)),
            pl.BlockSpec((EMBED, 2 * EMBED), lambda i: (0, 0)),
        ],
        out_specs=pl.BlockSpec((BMV, 2 * EMBED), lambda i: (i, 0)),
        out_shape=jax.ShapeDtypeStruct((VOCAB, 2 * EMBED), jnp.float32),
    )(tableT, WH2)


def _combine_chunk(gx_v, gy_v, buf, o_v, bvals):
    def body(r, _):
        for q in range(EMBED // 16):
            s = q * 16
            o_v[buf, r, pl.ds(s, 16)] = (
                gx_v[buf, r, pl.ds(s, 16)]
                + gy_v[buf, r, pl.ds(EMBED + s, 16)]
                + bvals[q])
        return 0

    lax.fori_loop(0, CHUNK, body, 0, unroll=4)


@functools.partial(
    pl.kernel,
    mesh=plsc.VectorSubcoreMesh(core_axis_name="c", subcore_axis_name="s"),
    out_type=jax.ShapeDtypeStruct((BATCH, EMBED), jnp.float32),
    scratch_types=[
        pltpu.VMEM((NCHUNK, CHUNK), jnp.int32),
        pltpu.VMEM((NCHUNK, CHUNK), jnp.int32),
        pltpu.VMEM((2, CHUNK, 2 * EMBED), jnp.float32),
        pltpu.VMEM((2, CHUNK, 2 * EMBED), jnp.float32),
        pltpu.VMEM((2, CHUNK, EMBED), jnp.float32),
        pltpu.VMEM((EMBED,), jnp.float32),
        pltpu.SemaphoreType.DMA,
        pltpu.SemaphoreType.DMA,
        pltpu.SemaphoreType.DMA,
    ],
    compiler_params=pltpu.CompilerParams(use_tc_tiling_on_sc=True),
)
def _gather_combine(idx_hbm, p_hbm, b_hbm, out_hbm, ix_v, iy_v, gx_v, gy_v,
                    o_v, b_v, isem, gsem, osem):
    wid = lax.axis_index("s") * NC + lax.axis_index("c")
    base = wid * PER_W
    pltpu.sync_copy(b_hbm, b_v)
    bvals = tuple(b_v[pl.ds(q * 16, 16)] for q in range(EMBED // 16))
    icp = [
        pltpu.async_copy(idx_hbm.at[pl.ds(base + j * CHUNK, CHUNK)],
                         ix_v.at[j], isem)
        for j in range(NCHUNK)
    ] + [
        pltpu.async_copy(idx_hbm.at[pl.ds(BATCH + base + j * CHUNK, CHUNK)],
                         iy_v.at[j], isem)
        for j in range(NCHUNK)
    ]
    for c in icp:
        c.wait()

    gets = [(pltpu.async_copy(p_hbm.at[ix_v.at[0]], gx_v.at[0], gsem),
             pltpu.async_copy(p_hbm.at[iy_v.at[0]], gy_v.at[0], gsem))]
    puts = []
    for j in range(NCHUNK):
        buf = j % 2
        if j + 1 < NCHUNK:
            nb = (j + 1) % 2
            gets.append(
                (pltpu.async_copy(p_hbm.at[ix_v.at[j + 1]], gx_v.at[nb], gsem),
                 pltpu.async_copy(p_hbm.at[iy_v.at[j + 1]], gy_v.at[nb], gsem)))
        gets[j][0].wait()
        gets[j][1].wait()
        if j >= 2:
            puts[j - 2].wait()
        _combine_chunk(gx_v, gy_v, buf, o_v, bvals)
        puts.append(pltpu.async_copy(
            o_v.at[buf], out_hbm.at[pl.ds(base + j * CHUNK, CHUNK)], osem))
    puts[NCHUNK - 2].wait()
    puts[NCHUNK - 1].wait()


def kernel(actions, table, W, b):
    idx = actions.astype(jnp.int32).T.reshape(2 * BATCH)
    WH2 = jnp.concatenate([W[:EMBED], W[EMBED:]], axis=1)
    P = _pmat(table.T, WH2)
    return _gather_combine(idx, P, b)


# speedup vs baseline: 1.2321x; 1.0219x over previous
import functools

import jax
import jax.numpy as jnp
from jax import lax
from jax.experimental import pallas as pl
from jax.experimental.pallas import tpu as pltpu
from jax.experimental.pallas import tpu_sc as plsc

EMBED = 64
BATCH = 16384
VOCAB = 100001
BMV = 16384

NC = 2
NS = 16
NW = NC * NS
PER_W = BATCH // NW
CHUNK = 128
NCHUNK = PER_W // CHUNK


def _pmat_body(tt_ref, wh_ref, p_ref):
    p_ref[...] = lax.dot_general(
        tt_ref[...], wh_ref[...], (((0,), (0,)), ((), ())),
        preferred_element_type=jnp.float32)


def _pmat(tableT, WH2):
    return pl.pallas_call(
        _pmat_body,
        grid=(pl.cdiv(VOCAB, BMV),),
        in_specs=[
            pl.BlockSpec((EMBED, BMV), lambda i: (0, i)),
            pl.BlockSpec((EMBED, 2 * EMBED), lambda i: (0, 0)),
        ],
        out_specs=pl.BlockSpec((BMV, 2 * EMBED), lambda i: (i, 0)),
        out_shape=jax.ShapeDtypeStruct((VOCAB, 2 * EMBED), jnp.float32),
    )(tableT, WH2)


def _combine_chunk(gx_v, gy_v, buf, o_v, bvals):
    def body(r, _):
        for q in range(EMBED // 16):
            s = q * 16
            o_v[buf, r, pl.ds(s, 16)] = (
                gx_v[buf, r, pl.ds(s, 16)]
                + gy_v[buf, r, pl.ds(EMBED + s, 16)]
                + bvals[q])
        return 0

    lax.fori_loop(0, CHUNK, body, 0, unroll=4)


@functools.partial(
    pl.kernel,
    mesh=plsc.VectorSubcoreMesh(core_axis_name="c", subcore_axis_name="s"),
    out_type=jax.ShapeDtypeStruct((BATCH, EMBED), jnp.float32),
    scratch_types=[
        pltpu.VMEM((NCHUNK, CHUNK), jnp.int32),
        pltpu.VMEM((NCHUNK, CHUNK), jnp.int32),
        pltpu.VMEM((2, CHUNK, 2 * EMBED), jnp.float32),
        pltpu.VMEM((2, CHUNK, 2 * EMBED), jnp.float32),
        pltpu.VMEM((2, CHUNK, EMBED), jnp.float32),
        pltpu.VMEM((EMBED,), jnp.float32),
        pltpu.SemaphoreType.DMA,
        pltpu.SemaphoreType.DMA,
        pltpu.SemaphoreType.DMA,
    ],
    compiler_params=pltpu.CompilerParams(use_tc_tiling_on_sc=True),
)
def _gather_combine(idx_hbm, p_hbm, b_hbm, out_hbm, ix_v, iy_v, gx_v, gy_v,
                    o_v, b_v, isem, gsem, osem):
    wid = lax.axis_index("s") * NC + lax.axis_index("c")
    base = wid * PER_W
    pltpu.sync_copy(b_hbm, b_v)
    bvals = tuple(b_v[pl.ds(q * 16, 16)] for q in range(EMBED // 16))
    icp = [
        pltpu.async_copy(idx_hbm.at[pl.ds(base + j * CHUNK, CHUNK)],
                         ix_v.at[j], isem)
        for j in range(NCHUNK)
    ] + [
        pltpu.async_copy(idx_hbm.at[pl.ds(BATCH + base + j * CHUNK, CHUNK)],
                         iy_v.at[j], isem)
        for j in range(NCHUNK)
    ]
    for c in icp:
        c.wait()

    gets = [(pltpu.async_copy(p_hbm.at[ix_v.at[0]], gx_v.at[0], gsem),
             pltpu.async_copy(p_hbm.at[iy_v.at[0]], gy_v.at[0], gsem))]
    puts = []
    for j in range(NCHUNK):
        buf = j % 2
        if j + 1 < NCHUNK:
            nb = (j + 1) % 2
            gets.append(
                (pltpu.async_copy(p_hbm.at[ix_v.at[j + 1]], gx_v.at[nb], gsem),
                 pltpu.async_copy(p_hbm.at[iy_v.at[j + 1]], gy_v.at[nb], gsem)))
        gets[j][0].wait()
        gets[j][1].wait()
        if j >= 2:
            puts[j - 2].wait()
        _combine_chunk(gx_v, gy_v, buf, o_v, bvals)
        puts.append(pltpu.async_copy(
            o_v.at[buf], out_hbm.at[pl.ds(base + j * CHUNK, CHUNK)], osem))
    puts[NCHUNK - 2].wait()
    puts[NCHUNK - 1].wait()


def kernel(actions, table, W, b):
    idx = actions.astype(jnp.int32).T.reshape(2 * BATCH)
    WH2 = jnp.concatenate([W[:EMBED], W[EMBED:]], axis=1)
    P = _pmat(table.T, WH2)
    return _gather_combine(idx, P, b)


# speedup vs baseline: 1.2424x; 1.0083x over previous
import functools

import jax
import jax.numpy as jnp
from jax import lax
from jax.experimental import pallas as pl
from jax.experimental.pallas import tpu as pltpu
from jax.experimental.pallas import tpu_sc as plsc

EMBED = 64
BATCH = 16384
VOCAB = 100001
BMV = 25088

NC = 2
NS = 16
NW = NC * NS
PER_W = BATCH // NW
CHUNK = 128
NCHUNK = PER_W // CHUNK


def _pmat_body(tt_ref, wh_ref, p_ref):
    p_ref[...] = lax.dot_general(
        tt_ref[...], wh_ref[...], (((0,), (0,)), ((), ())),
        preferred_element_type=jnp.float32)


def _pmat(tableT, WH2):
    return pl.pallas_call(
        _pmat_body,
        grid=(pl.cdiv(VOCAB, BMV),),
        in_specs=[
            pl.BlockSpec((EMBED, BMV), lambda i: (0, i)),
            pl.BlockSpec((EMBED, 2 * EMBED), lambda i: (0, 0)),
        ],
        out_specs=pl.BlockSpec((BMV, 2 * EMBED), lambda i: (i, 0)),
        out_shape=jax.ShapeDtypeStruct((VOCAB, 2 * EMBED), jnp.float32),
    )(tableT, WH2)


def _combine_chunk(gx_v, gy_v, buf, o_v, bvals):
    def body(r, _):
        for q in range(EMBED // 16):
            s = q * 16
            o_v[buf, r, pl.ds(s, 16)] = (
                gx_v[buf, r, pl.ds(s, 16)]
                + gy_v[buf, r, pl.ds(EMBED + s, 16)]
                + bvals[q])
        return 0

    lax.fori_loop(0, CHUNK, body, 0, unroll=4)


@functools.partial(
    pl.kernel,
    mesh=plsc.VectorSubcoreMesh(core_axis_name="c", subcore_axis_name="s"),
    out_type=jax.ShapeDtypeStruct((BATCH, EMBED), jnp.float32),
    scratch_types=[
        pltpu.VMEM((NCHUNK, CHUNK), jnp.int32),
        pltpu.VMEM((NCHUNK, CHUNK), jnp.int32),
        pltpu.VMEM((2, CHUNK, 2 * EMBED), jnp.float32),
        pltpu.VMEM((2, CHUNK, 2 * EMBED), jnp.float32),
        pltpu.VMEM((2, CHUNK, EMBED), jnp.float32),
        pltpu.VMEM((EMBED,), jnp.float32),
        pltpu.SemaphoreType.DMA,
        pltpu.SemaphoreType.DMA,
        pltpu.SemaphoreType.DMA,
    ],
    compiler_params=pltpu.CompilerParams(use_tc_tiling_on_sc=True),
)
def _gather_combine(idx_hbm, p_hbm, b_hbm, out_hbm, ix_v, iy_v, gx_v, gy_v,
                    o_v, b_v, isem, gsem, osem):
    wid = lax.axis_index("s") * NC + lax.axis_index("c")
    base = wid * PER_W
    pltpu.sync_copy(b_hbm, b_v)
    bvals = tuple(b_v[pl.ds(q * 16, 16)] for q in range(EMBED // 16))
    icp = [
        pltpu.async_copy(idx_hbm.at[pl.ds(base + j * CHUNK, CHUNK)],
                         ix_v.at[j], isem)
        for j in range(NCHUNK)
    ] + [
        pltpu.async_copy(idx_hbm.at[pl.ds(BATCH + base + j * CHUNK, CHUNK)],
                         iy_v.at[j], isem)
        for j in range(NCHUNK)
    ]
    for c in icp:
        c.wait()

    gets = [(pltpu.async_copy(p_hbm.at[ix_v.at[0]], gx_v.at[0], gsem),
             pltpu.async_copy(p_hbm.at[iy_v.at[0]], gy_v.at[0], gsem))]
    puts = []
    for j in range(NCHUNK):
        buf = j % 2
        if j + 1 < NCHUNK:
            nb = (j + 1) % 2
            gets.append(
                (pltpu.async_copy(p_hbm.at[ix_v.at[j + 1]], gx_v.at[nb], gsem),
                 pltpu.async_copy(p_hbm.at[iy_v.at[j + 1]], gy_v.at[nb], gsem)))
        gets[j][0].wait()
        gets[j][1].wait()
        if j >= 2:
            puts[j - 2].wait()
        _combine_chunk(gx_v, gy_v, buf, o_v, bvals)
        puts.append(pltpu.async_copy(
            o_v.at[buf], out_hbm.at[pl.ds(base + j * CHUNK, CHUNK)], osem))
    puts[NCHUNK - 2].wait()
    puts[NCHUNK - 1].wait()


def kernel(actions, table, W, b):
    idx = actions.astype(jnp.int32).T.reshape(2 * BATCH)
    WH2 = jnp.concatenate([W[:EMBED], W[EMBED:]], axis=1)
    P = _pmat(table.T, WH2)
    return _gather_combine(idx, P, b)
